# baseline (speedup 1.0000x reference)
import jax
import jax.numpy as jnp
from jax.experimental import pallas as pl
from jax.experimental.pallas import tpu as pltpu
from jax.experimental.pallas import tpu_sc as plsc

D = 64
GATHER_WINDOW = 256
TC_BLOCK = 2048


def _sc_gather(table, idx_flat):
    n = idx_flat.shape[1]
    mesh = plsc.VectorSubcoreMesh(core_axis_name="core",
                                  subcore_axis_name="subcore")

    @pl.kernel(
        out_type=jax.ShapeDtypeStruct((n, D), table.dtype),
        mesh=mesh,
        compiler_params=pltpu.CompilerParams(use_tc_tiling_on_sc=False),
    )
    def gather_kernel(table_hbm, idx_hbm, out_hbm):
        def body(idx_vmem, out_vmem):
            pltpu.sync_copy(table_hbm.at[idx_vmem.at[0]], out_vmem)

        pltpu.emit_pipeline(
            body,
            grid=(n // GATHER_WINDOW,),
            in_specs=[pl.BlockSpec((1, GATHER_WINDOW),
                                   index_map=lambda i: (0, i))],
            out_specs=[pl.BlockSpec((GATHER_WINDOW, D),
                                    index_map=lambda i: (i, 0))],
            core_axis_name=("core", "subcore"),
            dimension_semantics=(pltpu.PARALLEL,),
        )(idx_hbm, out_hbm)

    return gather_kernel(table, idx_flat)


def _tc_linear_tanh(emb_lbd, W, b):
    L, B, _ = emb_lbd.shape

    def body(emb_ref, w_ref, b_ref, o_ref):
        e = emb_ref[0]
        y = jax.lax.dot_general(
            w_ref[...], e,
            dimension_numbers=(((1,), (1,)), ((), ())),
            preferred_element_type=jnp.float32,
        )
        o_ref[0] = jnp.tanh(y + b_ref[...])

    return pl.pallas_call(
        body,
        grid=(L, B // TC_BLOCK),
        in_specs=[
            pl.BlockSpec((1, TC_BLOCK, D), lambda l, j: (l, j, 0)),
            pl.BlockSpec((D, D), lambda l, j: (0, 0)),
            pl.BlockSpec((D, 1), lambda l, j: (0, 0)),
        ],
        out_specs=pl.BlockSpec((1, D, TC_BLOCK), lambda l, j: (l, 0, j)),
        out_shape=jax.ShapeDtypeStruct((L, D, B), jnp.float32),
    )(emb_lbd, W, b.reshape(D, 1))


def kernel(x, table, W, b):
    B, L = x.shape
    n = B * L
    idx_flat = x.T.reshape(1, n).astype(jnp.int32)
    emb = _sc_gather(table, idx_flat)
    emb_lbd = emb.reshape(L, B, D)
    out_ldb = _tc_linear_tanh(emb_lbd, W, b)
    return out_ldb.transpose(2, 0, 1)


# speedup vs baseline: 1.1409x; 1.1409x over previous
import jax
import jax.numpy as jnp
from jax.experimental import pallas as pl
from jax.experimental.pallas import tpu as pltpu
from jax.experimental.pallas import tpu_sc as plsc

D = 64
GATHER_WINDOW = 256


def _tc_flatten_idx(x_t):
    L, B = x_t.shape

    def body(x_ref, o_ref, sem):
        for l in range(L):
            pltpu.make_async_copy(
                x_ref.at[l], o_ref.at[pl.ds(l * B, B)], sem.at[l]
            ).start()
        for l in range(L):
            pltpu.make_async_copy(
                x_ref.at[l], o_ref.at[pl.ds(l * B, B)], sem.at[l]
            ).wait()

    return pl.pallas_call(
        body,
        in_specs=[pl.BlockSpec(memory_space=pl.ANY)],
        out_specs=pl.BlockSpec(memory_space=pl.ANY),
        out_shape=jax.ShapeDtypeStruct((L * B,), x_t.dtype),
        scratch_shapes=[pltpu.SemaphoreType.DMA((L,))],
    )(x_t)


def _sc_gather(table, idx_flat):
    n = idx_flat.shape[1]
    mesh = plsc.VectorSubcoreMesh(core_axis_name="core",
                                  subcore_axis_name="subcore")

    @pl.kernel(
        out_type=jax.ShapeDtypeStruct((n, D), table.dtype),
        mesh=mesh,
        compiler_params=pltpu.CompilerParams(use_tc_tiling_on_sc=False),
    )
    def gather_kernel(table_hbm, idx_hbm, out_hbm):
        def body(idx_vmem, out_vmem):
            pltpu.sync_copy(table_hbm.at[idx_vmem.at[0]], out_vmem)

        pltpu.emit_pipeline(
            body,
            grid=(n // GATHER_WINDOW,),
            in_specs=[pl.BlockSpec((1, GATHER_WINDOW),
                                   index_map=lambda i: (0, i))],
            out_specs=[pl.BlockSpec((GATHER_WINDOW, D),
                                    index_map=lambda i: (i, 0))],
            core_axis_name=("core", "subcore"),
            dimension_semantics=(pltpu.PARALLEL,),
        )(idx_hbm, out_hbm)

    return gather_kernel(table, idx_flat)


def _tc_linear_tanh(emb_lbd, W, b):
    L, B, _ = emb_lbd.shape

    def body(emb_ref, w_ref, b_ref, o_ref):
        e = emb_ref[0]
        y = jax.lax.dot_general(
            w_ref[...], e,
            dimension_numbers=(((1,), (1,)), ((), ())),
            preferred_element_type=jnp.float32,
        )
        o_ref[0] = jnp.tanh(y + b_ref[...])

    return pl.pallas_call(
        body,
        grid=(L,),
        in_specs=[
            pl.BlockSpec((1, B, D), lambda l: (l, 0, 0)),
            pl.BlockSpec((D, D), lambda l: (0, 0)),
            pl.BlockSpec((D, 1), lambda l: (0, 0)),
        ],
        out_specs=pl.BlockSpec((1, D, B), lambda l: (l, 0, 0)),
        out_shape=jax.ShapeDtypeStruct((L, D, B), jnp.float32),
    )(emb_lbd, W, b.reshape(D, 1))


def kernel(x, table, W, b):
    B, L = x.shape
    n = B * L
    idx_flat = _tc_flatten_idx(x.T.astype(jnp.int32)).reshape(1, n)
    emb = _sc_gather(table, idx_flat)
    emb_lbd = emb.reshape(L, B, D)
    out_ldb = _tc_linear_tanh(emb_lbd, W, b)
    return out_ldb.transpose(2, 0, 1)


# speedup vs baseline: 2.1222x; 1.8602x over previous
import jax
import jax.numpy as jnp
from jax.experimental import pallas as pl
from jax.experimental.pallas import tpu as pltpu
from jax.experimental.pallas import tpu_sc as plsc

D = 64
H = 524288
REPACK_BLK = 4096
GATHER_WINDOW = 256


def _tc_flatten_idx(x_t, h):
    L, B = x_t.shape

    def body(x_ref, o_ref, scratch, sem):
        l = pl.program_id(0)
        cp = pltpu.make_async_copy(x_ref.at[l], scratch, sem)
        cp.start()
        cp.wait()
        i = scratch[...]
        o_ref[...] = jnp.where(i < h, 2 * i, 2 * i - (2 * h - 1))

    return pl.pallas_call(
        body,
        grid=(L,),
        in_specs=[pl.BlockSpec(memory_space=pl.ANY)],
        out_specs=pl.BlockSpec((B,), lambda l: (l,)),
        out_shape=jax.ShapeDtypeStruct((L * B,), x_t.dtype),
        scratch_shapes=[pltpu.VMEM((B,), x_t.dtype),
                        pltpu.SemaphoreType.DMA],
    )(x_t)


def _tc_repack_table(table_t, eye):
    V = table_t.shape[1]
    n_in_blocks = pl.cdiv(V, REPACK_BLK)
    half_blocks = H // REPACK_BLK

    def body(t1_ref, t2_ref, eye_ref, o_ref):
        t1 = jax.lax.dot_general(
            t1_ref[...], eye_ref[...],
            dimension_numbers=(((0,), (0,)), ((), ())),
            preferred_element_type=jnp.float32,
        )
        t2 = jax.lax.dot_general(
            t2_ref[...], eye_ref[...],
            dimension_numbers=(((0,), (0,)), ((), ())),
            preferred_element_type=jnp.float32,
        )
        o_ref[:, :D] = t1
        o_ref[:, D:] = t2

    return pl.pallas_call(
        body,
        grid=(half_blocks,),
        in_specs=[
            pl.BlockSpec((D, REPACK_BLK), lambda j: (0, j)),
            pl.BlockSpec((D, REPACK_BLK),
                         lambda j: (0, jnp.minimum(j + half_blocks,
                                                   n_in_blocks - 1))),
            pl.BlockSpec((D, D), lambda j: (0, 0)),
        ],
        out_specs=pl.BlockSpec((REPACK_BLK, 2 * D), lambda j: (j, 0)),
        out_shape=jax.ShapeDtypeStruct((H, 2 * D), jnp.float32),
    )(table_t, table_t, eye)


def _sc_gather(table_lin, idx_flat):
    n = idx_flat.shape[1]
    mesh = plsc.VectorSubcoreMesh(core_axis_name="core",
                                  subcore_axis_name="subcore")

    @pl.kernel(
        out_type=jax.ShapeDtypeStruct((n, 2 * D), table_lin.dtype),
        mesh=mesh,
        compiler_params=pltpu.CompilerParams(use_tc_tiling_on_sc=False),
    )
    def gather_kernel(table_hbm, idx_hbm, out_hbm):
        def body(idx_vmem, out_vmem):
            pltpu.sync_copy(table_hbm.at[idx_vmem.at[0]], out_vmem)

        pltpu.emit_pipeline(
            body,
            grid=(n // GATHER_WINDOW,),
            in_specs=[pl.BlockSpec((1, GATHER_WINDOW),
                                   index_map=lambda i: (0, i))],
            out_specs=[pl.BlockSpec((GATHER_WINDOW, D),
                                    index_map=lambda i: (i, 0))],
            core_axis_name=("core", "subcore"),
            dimension_semantics=(pltpu.PARALLEL,),
        )(idx_hbm, out_hbm)

    return gather_kernel(table_lin, idx_flat)


def _tc_linear_tanh(emb_pad, W, b, L, B):
    def body(emb_ref, w_ref, b_ref, o_ref):
        e = emb_ref[0, :, :D]
        y = jax.lax.dot_general(
            w_ref[...], e,
            dimension_numbers=(((1,), (1,)), ((), ())),
            preferred_element_type=jnp.float32,
        )
        o_ref[0] = jnp.tanh(y + b_ref[...])

    return pl.pallas_call(
        body,
        grid=(L,),
        in_specs=[
            pl.BlockSpec((1, B, 2 * D), lambda l: (l, 0, 0)),
            pl.BlockSpec((D, D), lambda l: (0, 0)),
            pl.BlockSpec((D, 1), lambda l: (0, 0)),
        ],
        out_specs=pl.BlockSpec((1, D, B), lambda l: (l, 0, 0)),
        out_shape=jax.ShapeDtypeStruct((L, D, B), jnp.float32),
    )(emb_pad, W, b.reshape(D, 1))


def kernel(x, table, W, b):
    B, L = x.shape
    n = B * L
    idx_flat = _tc_flatten_idx(x.T.astype(jnp.int32), H)
    packed = _tc_repack_table(table.T, jnp.eye(D, dtype=jnp.float32))
    table_lin = packed.reshape(2 * H, D)
    emb = _sc_gather(table_lin, idx_flat.reshape(1, n))
    emb_pad = emb.reshape(L, B, 2 * D)
    out_ldb = _tc_linear_tanh(emb_pad, W, b, L, B)
    return out_ldb.transpose(2, 0, 1)
